# Initial kernel scaffold; baseline (speedup 1.0000x reference)
import jax
import jax.numpy as jnp
from jax.experimental import pallas as pl


def kernel(x, sdf_grid):
    raise NotImplementedError("write your pallas kernel here")


# speedup vs baseline: 2.1731x; 2.1731x over previous
import functools

import jax
import jax.numpy as jnp
from jax import lax
from jax.experimental import pallas as pl
from jax.experimental.pallas import tpu as pltpu
from jax.experimental.pallas import tpu_sc as plsc

XMIN, XMAX, YMIN, YMAX = -1.0, 1.0, -1.0, 1.0

_C = 2048
_NW = 32
_L = 16


def _floor_parts(v):
    ti = v.astype(jnp.int32)
    tf = ti.astype(jnp.float32)
    adj = tf > v
    fi = jnp.where(adj, ti - 1, ti)
    ff = jnp.where(adj, tf - 1.0, tf)
    return fi, ff, v - ff


def _rsqrt(v):
    i = lax.bitcast_convert_type(v, jnp.int32)
    i = jnp.int32(0x5F3759DF) - lax.shift_right_arithmetic(i, 1)
    y = lax.bitcast_convert_type(i, jnp.float32)
    h = 0.5 * v
    for _ in range(3):
        y = y * (1.5 - h * y * y)
    return y


def _sc_body(Hh, Ww, npts, nchunks, x0_hbm, x1_hbm, grid_hbm,
             out0_hbm, out1_hbm, x0buf, x1buf, idxbufs, valbufs, out0buf,
             out1buf, sem):
    wid = lax.axis_index("s") * 2 + lax.axis_index("c")

    nvec = _C // _L

    fWm1 = float(Ww - 1)
    fHm1 = float(Hh - 1)
    sx = 0.5 * float(Ww - 1)
    sy = 0.5 * float(Hh - 1)

    def coords(j):
        base = j * _L
        gx_raw = x0buf[pl.ds(base, _L)]
        gy_raw = x1buf[pl.ds(base, _L)]
        gx = 2.0 * (gx_raw - XMIN) / (XMAX - XMIN) - 1.0
        gy = 2.0 * (gy_raw - YMIN) / (YMAX - YMIN) - 1.0
        ix = (gx + 1.0) * sx
        iy = (gy + 1.0) * sy
        ix0, ix0f, wx = _floor_parts(ix)
        iy0, iy0f, wy = _floor_parts(iy)
        return gx_raw, gy_raw, ix, iy, ix0, ix0f, wx, iy0, iy0f, wy

    def cell_cols(ix0):
        return [jnp.clip(ix0 + d, 0, Ww - 1) for d in (-1, 0, 1, 2)]

    def cell_rows(iy0):
        return [jnp.clip(iy0 + d, 0, Hh - 1) for d in (-1, 0, 1, 2)]

    cells = [(-1, 0), (-1, 1),
             (0, -1), (0, 0), (0, 1), (0, 2),
             (1, -1), (1, 0), (1, 1), (1, 2),
             (2, 0), (2, 1)]

    def pass1(j, _):
        (_, _, _, _, ix0, _, _, iy0, _, _) = coords(j)
        cols = cell_cols(ix0)
        rows = cell_rows(iy0)
        off = j * _L
        for k, (dr, dc) in enumerate(cells):
            idx = rows[dr + 1] * Ww + cols[dc + 1]
            idxbufs[k][pl.ds(off, _L)] = idx
        return 0

    def pass2(j, _):
        (gx_raw, gy_raw, ix, iy, ix0, ix0f, wx, iy0, iy0f, wy) = coords(j)
        off = j * _L

        v = [valbufs[k][pl.ds(off, _L)] for k in range(len(cells))]

        colok = [((ix0 + d) >= 0) & ((ix0 + d) <= (Ww - 1)) for d in (-1, 0, 1, 2)]
        rowok = [((iy0 + d) >= 0) & ((iy0 + d) <= (Hh - 1)) for d in (-1, 0, 1, 2)]
        vm = [jnp.where(rowok[dr + 1] & colok[dc + 1], v[k], 0.0)
              for k, (dr, dc) in enumerate(cells)]

        (A, B, Cc, D, E, F, G, Hc, I, J, K, Lc) = vm
        ixc = jnp.clip(ix, 0.0, fWm1)
        iyc = jnp.clip(iy, 0.0, fHm1)
        wxc = ixc - jnp.clip(ix0f, 0.0, fWm1)
        wyc = iyc - jnp.clip(iy0f, 0.0, fHm1)
        Dr, Er, Hr, Ir = v[3], v[4], v[7], v[8]
        phi = ((1.0 - wyc) * ((1.0 - wxc) * Dr + wxc * Er)
               + wyc * ((1.0 - wxc) * Hr + wxc * Ir))

        omx = 1.0 - wx
        omy = 1.0 - wy
        gx_p = omy * (omx * E + wx * F) + wy * (omx * I + wx * J)
        gx_m = omy * (omx * Cc + wx * D) + wy * (omx * G + wx * Hc)
        gy_p = omy * (omx * Hc + wx * I) + wy * (omx * K + wx * Lc)
        gy_m = omy * (omx * A + wx * B) + wy * (omx * D + wx * E)

        dphidx = (gx_p - gx_m) * (fWm1 / 4.0)
        dphidy = (gy_p - gy_m) * (fHm1 / 4.0)
        g2 = jnp.maximum(dphidx * dphidx + dphidy * dphidy, 1e-12)
        coef = phi / g2 + 0.001 * _rsqrt(g2)
        newx = gx_raw - coef * dphidx
        newy = gy_raw - coef * dphidy
        sel = phi > 0.0
        base = j * _L
        out0buf[pl.ds(base, _L)] = jnp.where(sel, newx, gx_raw)
        out1buf[pl.ds(base, _L)] = jnp.where(sel, newy, gy_raw)
        return 0

    def chunk(t, _):
        i = jnp.minimum(wid + _NW * t, nchunks - 1)
        start = pl.multiple_of(jnp.minimum(i * _C, npts - _C), 8)
        pltpu.sync_copy(x0_hbm.at[pl.ds(start, _C)], x0buf)
        pltpu.sync_copy(x1_hbm.at[pl.ds(start, _C)], x1buf)
        lax.fori_loop(0, nvec, pass1, 0)
        descs = [
            pltpu.async_copy(grid_hbm.at[idxbufs[k]], valbufs[k], sem)
            for k in range(len(cells))
        ]
        for d in descs:
            d.wait()
        lax.fori_loop(0, nvec, pass2, 0)
        pltpu.sync_copy(out0buf, out0_hbm.at[pl.ds(start, _C)])
        pltpu.sync_copy(out1buf, out1_hbm.at[pl.ds(start, _C)])
        return 0

    t_per_w = -(-nchunks // _NW)
    lax.fori_loop(0, t_per_w, chunk, 0)


def kernel(x, sdf_grid):
    npts = x.shape[0]
    Hh, Ww = sdf_grid.shape[2], sdf_grid.shape[3]
    ncell = 12

    assert npts % 8 == 0 and npts >= _C
    nchunks = -(-npts // _C)

    xt = x.T
    x0_flat = xt[0]
    x1_flat = xt[1]
    grid_flat = sdf_grid.reshape(Hh * Ww)

    mesh = plsc.VectorSubcoreMesh(core_axis_name="c", subcore_axis_name="s")
    body = functools.partial(_sc_body, Hh, Ww, npts, nchunks)
    out0, out1 = pl.kernel(
        body,
        out_type=(
            jax.ShapeDtypeStruct((npts,), jnp.float32),
            jax.ShapeDtypeStruct((npts,), jnp.float32),
        ),
        mesh=mesh,
        scratch_types=[
            pltpu.VMEM((_C,), jnp.float32),
            pltpu.VMEM((_C,), jnp.float32),
            [pltpu.VMEM((_C,), jnp.int32) for _ in range(ncell)],
            [pltpu.VMEM((_C,), jnp.float32) for _ in range(ncell)],
            pltpu.VMEM((_C,), jnp.float32),
            pltpu.VMEM((_C,), jnp.float32),
            pltpu.SemaphoreType.DMA,
        ],
    )(x0_flat, x1_flat, grid_flat)
    return jnp.stack([out0, out1], axis=1)
